# baseline (speedup 1.0000x reference)
import functools

import jax
import jax.numpy as jnp
from jax import lax
from jax.experimental import pallas as pl
from jax.experimental.pallas import tpu as pltpu
from jax.experimental.pallas import tpu_sc as plsc

BATCH = 16384
DIM = 32
LANES = 16

_info = plsc.get_sparse_core_info()
NC, NS = _info.num_cores, _info.num_subcores
NW = NC * NS
B_PER_W = BATCH // NW
CHUNK = 128
N_CHUNKS = B_PER_W // CHUNK
GROUPS = B_PER_W // LANES


def _gmf_body(users_hbm, items_hbm, ut_hbm, it_hbm, w_hbm, b_hbm, out_hbm,
              uidx_v, iidx_v, urows_v, irows_v, w_v, b_v, tile_v, out_v, sem):
    wid = lax.axis_index("s") * NC + lax.axis_index("c")
    base = wid * B_PER_W

    pltpu.sync_copy(users_hbm.at[pl.ds(base, B_PER_W)], uidx_v)
    pltpu.sync_copy(items_hbm.at[pl.ds(base, B_PER_W)], iidx_v)
    pltpu.sync_copy(w_hbm, w_v)
    pltpu.sync_copy(b_hbm, b_v)

    copies = []
    for c in range(N_CHUNKS):
        sl = pl.ds(c * CHUNK, CHUNK)
        copies.append(pltpu.async_copy(ut_hbm.at[uidx_v.at[sl]], urows_v.at[sl], sem))
        copies.append(pltpu.async_copy(it_hbm.at[iidx_v.at[sl]], irows_v.at[sl], sem))
    for cp in copies:
        cp.wait()

    w0 = w_v[pl.ds(0, LANES)]
    w1 = w_v[pl.ds(LANES, LANES)]
    bias = b_v[...]
    lane = lax.broadcasted_iota(jnp.int32, (LANES,), 0)

    def group_body(g, _):
        row0 = g * LANES
        for r in range(LANES):
            row = row0 + r
            ua = urows_v[row, pl.ds(0, LANES)]
            ub = urows_v[row, pl.ds(LANES, LANES)]
            ia = irows_v[row, pl.ds(0, LANES)]
            ib = irows_v[row, pl.ds(LANES, LANES)]
            part = ua * ia * w0 + ub * ib * w1
            plsc.store_scatter(tile_v, [lane, jnp.full((LANES,), r, jnp.int32)], part)
        acc = tile_v[0, pl.ds(0, LANES)]
        for l in range(1, LANES):
            acc = acc + tile_v[l, pl.ds(0, LANES)]
        logit = acc + bias
        prob = 1.0 / (1.0 + jnp.exp(-logit))
        out_v[pl.ds(pl.multiple_of(row0, LANES), LANES)] = prob
        return 0

    lax.fori_loop(0, GROUPS, group_body, 0)

    pltpu.sync_copy(out_v, out_hbm.at[pl.ds(base, B_PER_W)])


@functools.partial(jax.jit, static_argnames=())
def _gmf_call(users, items, user_table, item_table, w_flat, b_vec):
    mesh = plsc.VectorSubcoreMesh(core_axis_name="c", subcore_axis_name="s")
    kern = functools.partial(
        pl.kernel,
        out_type=jax.ShapeDtypeStruct((BATCH,), jnp.float32),
        mesh=mesh,
        compiler_params=pltpu.CompilerParams(
            needs_layout_passes=False, use_tc_tiling_on_sc=False),
        scratch_types=[
            pltpu.VMEM((B_PER_W,), jnp.int32),
            pltpu.VMEM((B_PER_W,), jnp.int32),
            pltpu.VMEM((B_PER_W, DIM), jnp.float32),
            pltpu.VMEM((B_PER_W, DIM), jnp.float32),
            pltpu.VMEM((DIM,), jnp.float32),
            pltpu.VMEM((LANES,), jnp.float32),
            pltpu.VMEM((LANES, LANES), jnp.float32),
            pltpu.VMEM((B_PER_W,), jnp.float32),
            pltpu.SemaphoreType.DMA,
        ],
    )(_gmf_body)
    return kern(users, items, user_table, item_table, w_flat, b_vec)


def kernel(users, items, user_table, item_table, W, b):
    w_flat = W.reshape(DIM).astype(jnp.float32)
    b_vec = jnp.broadcast_to(b.reshape(()), (LANES,)).astype(jnp.float32)
    out = _gmf_call(users, items, user_table, item_table, w_flat, b_vec)
    return out.reshape(BATCH, 1)
